# baseline (speedup 1.0000x reference)
import jax
import jax.numpy as jnp
from jax.experimental import pallas as pl

_N_TOKENS = 32768
_D_IN = 1024
_E = 8
_W_B = 256
_W_R = 128
_BT = 2048


def _moe_kernel(x_ref, w1_ref, b1_ref, w2_ref, b2_ref, w3_ref, b3_ref,
                rw1_ref, rb1_ref, rw2_ref, rb2_ref, out_ref):
    xb = x_ref[:].astype(jnp.bfloat16)

    rh = jnp.maximum(
        jnp.dot(xb, rw1_ref[:], preferred_element_type=jnp.float32)
        .astype(jnp.bfloat16) + rb1_ref[:], jnp.bfloat16(0.0))
    logits = (jnp.dot(rh, rw2_ref[:], preferred_element_type=jnp.float32)
              + rb2_ref[:])
    m = jnp.max(logits, axis=-1, keepdims=True)
    eg = jnp.exp(logits - m)
    gate = eg / jnp.sum(eg, axis=-1, keepdims=True)

    gate_bf = gate.astype(jnp.bfloat16)
    cols = []
    for e in range(_E):
        h1e = jnp.maximum(
            jnp.dot(xb, w1_ref[e], preferred_element_type=jnp.float32)
            .astype(jnp.bfloat16) + b1_ref[e:e + 1, :],
            jnp.bfloat16(0.0))
        h2e = jnp.maximum(
            jnp.dot(h1e, w2_ref[e], preferred_element_type=jnp.float32)
            .astype(jnp.bfloat16) + b2_ref[e:e + 1, :], jnp.bfloat16(0.0))
        cols.append(gate_bf[:, e:e + 1] * h2e)
    hcat = jnp.concatenate(cols, axis=1)
    y = jnp.dot(hcat, w3_ref[:], preferred_element_type=jnp.float32)
    y = y + jnp.dot(gate_bf, b3_ref[:],
                    preferred_element_type=jnp.float32)
    out_ref[:] = y


def kernel(x, W1, b1, W2, b2, W3, b3, Rw1, Rb1, Rw2, Rb2):
    w1_bf = W1.astype(jnp.bfloat16)
    b1_bf = b1.astype(jnp.bfloat16)
    w2_bf = W2.astype(jnp.bfloat16)
    b2_bf = b2.astype(jnp.bfloat16)
    w3_flat = W3.reshape(_E * _W_B, 1).astype(jnp.bfloat16)
    b3_bf = b3.astype(jnp.bfloat16)
    rw1_bf = Rw1.astype(jnp.bfloat16)
    rw2_bf = Rw2.astype(jnp.bfloat16)
    rb1_2d = Rb1.reshape(1, _W_R).astype(jnp.bfloat16)
    rb2_2d = Rb2.reshape(1, _E)

    grid = (_N_TOKENS // _BT,)
    full = lambda *shape: pl.BlockSpec(shape, lambda i: (0,) * len(shape))
    out = pl.pallas_call(
        _moe_kernel,
        grid=grid,
        in_specs=[
            pl.BlockSpec((_BT, _D_IN), lambda i: (i, 0)),
            full(_E, _D_IN, _W_B),
            full(_E, _W_B),
            full(_E, _W_B, _W_B),
            full(_E, _W_B),
            full(_E * _W_B, 1),
            full(_E, 1),
            full(_D_IN, _W_R),
            full(1, _W_R),
            full(_W_R, _E),
            full(1, _E),
        ],
        out_specs=pl.BlockSpec((_BT, 1), lambda i: (i, 0)),
        out_shape=jax.ShapeDtypeStruct((_N_TOKENS, 1), jnp.float32),
    )(x, w1_bf, b1_bf, w2_bf, b2_bf, w3_flat, b3_bf,
      rw1_bf, rb1_2d, rw2_bf, rb2_2d)
    return out


# speedup vs baseline: 1.5923x; 1.5923x over previous
import jax
import jax.numpy as jnp
from jax.experimental import pallas as pl

_N_TOKENS = 32768
_D_IN = 1024
_E = 8
_W_B = 256
_W_R = 128
_BT = 2048
_EC = 4


def _moe_kernel(x_ref, w1_ref, b1_ref, w2_ref, b2_ref, w3_ref, b3_ref,
                rw1_ref, rb1_ref, rw2_ref, rb2_ref, out_ref):
    xb = x_ref[:].astype(jnp.bfloat16)

    rh = jnp.maximum(
        jnp.dot(xb, rw1_ref[:], preferred_element_type=jnp.float32)
        .astype(jnp.bfloat16) + rb1_ref[:], jnp.bfloat16(0.0))
    logits = (jnp.dot(rh, rw2_ref[:], preferred_element_type=jnp.float32)
              + rb2_ref[:])
    m = jnp.max(logits, axis=-1, keepdims=True)
    eg = jnp.exp(logits - m)
    gate = eg / jnp.sum(eg, axis=-1, keepdims=True)
    gate_bf = gate.astype(jnp.bfloat16)

    cw = _EC * _W_B
    cols = []
    for c in range(_E // _EC):
        h1c = jnp.maximum(
            jnp.dot(xb, w1_ref[:, c * cw:(c + 1) * cw],
                    preferred_element_type=jnp.float32)
            .astype(jnp.bfloat16) + b1_ref[:, c * cw:(c + 1) * cw],
            jnp.bfloat16(0.0))
        for k in range(_EC):
            e = c * _EC + k
            h1e = h1c[:, k * _W_B:(k + 1) * _W_B]
            h2e = jnp.maximum(
                jnp.dot(h1e, w2_ref[e], preferred_element_type=jnp.float32)
                .astype(jnp.bfloat16) + b2_ref[e:e + 1, :],
                jnp.bfloat16(0.0))
            cols.append(gate_bf[:, e:e + 1] * h2e)
    hcat = jnp.concatenate(cols, axis=1)
    y = jnp.dot(hcat, w3_ref[:], preferred_element_type=jnp.float32)
    y = y + jnp.dot(gate_bf, b3_ref[:],
                    preferred_element_type=jnp.float32)
    out_ref[:] = y


def kernel(x, W1, b1, W2, b2, W3, b3, Rw1, Rb1, Rw2, Rb2):
    w1_flat = W1.transpose(1, 0, 2).reshape(_D_IN, _E * _W_B)
    w1_flat = w1_flat.astype(jnp.bfloat16)
    b1_flat = b1.reshape(1, _E * _W_B).astype(jnp.bfloat16)
    w2_bf = W2.astype(jnp.bfloat16)
    b2_bf = b2.astype(jnp.bfloat16)
    w3_flat = W3.reshape(_E * _W_B, 1).astype(jnp.bfloat16)
    b3_bf = b3.astype(jnp.bfloat16)
    rw1_bf = Rw1.astype(jnp.bfloat16)
    rw2_bf = Rw2.astype(jnp.bfloat16)
    rb1_2d = Rb1.reshape(1, _W_R).astype(jnp.bfloat16)
    rb2_2d = Rb2.reshape(1, _E)

    grid = (_N_TOKENS // _BT,)
    full = lambda *shape: pl.BlockSpec(shape, lambda i: (0,) * len(shape))
    out = pl.pallas_call(
        _moe_kernel,
        grid=grid,
        in_specs=[
            pl.BlockSpec((_BT, _D_IN), lambda i: (i, 0)),
            full(_D_IN, _E * _W_B),
            full(1, _E * _W_B),
            full(_E, _W_B, _W_B),
            full(_E, _W_B),
            full(_E * _W_B, 1),
            full(_E, 1),
            full(_D_IN, _W_R),
            full(1, _W_R),
            full(_W_R, _E),
            full(1, _E),
        ],
        out_specs=pl.BlockSpec((_BT, 1), lambda i: (i, 0)),
        out_shape=jax.ShapeDtypeStruct((_N_TOKENS, 1), jnp.float32),
    )(x, w1_flat, b1_flat, w2_bf, b2_bf, w3_flat, b3_bf,
      rw1_bf, rb1_2d, rw2_bf, rb2_2d)
    return out


# speedup vs baseline: 1.6065x; 1.0089x over previous
import jax
import jax.numpy as jnp
from jax.experimental import pallas as pl

_N_TOKENS = 32768
_D_IN = 1024
_E = 8
_W_B = 256
_W_R = 128
_BT = 2048
_EC = 2


def _moe_kernel(x_ref, w1_ref, b1_ref, w2_ref, b2_ref, w3_ref, b3_ref,
                rw1_ref, rb1_ref, rw2_ref, rb2_ref, out_ref):
    xb = x_ref[:].astype(jnp.bfloat16)

    rh = jnp.maximum(
        jnp.dot(xb, rw1_ref[:], preferred_element_type=jnp.float32)
        .astype(jnp.bfloat16) + rb1_ref[:], jnp.bfloat16(0.0))
    logits = (jnp.dot(rh, rw2_ref[:], preferred_element_type=jnp.float32)
              + rb2_ref[:])
    m = jnp.max(logits, axis=-1, keepdims=True)
    eg = jnp.exp(logits - m)
    gate = eg / jnp.sum(eg, axis=-1, keepdims=True)
    gate_bf = gate.astype(jnp.bfloat16)

    cw = _EC * _W_B
    cols = []
    for c in range(_E // _EC):
        h1c = jnp.maximum(
            jnp.dot(xb, w1_ref[:, c * cw:(c + 1) * cw],
                    preferred_element_type=jnp.float32)
            .astype(jnp.bfloat16) + b1_ref[:, c * cw:(c + 1) * cw],
            jnp.bfloat16(0.0))
        for k in range(_EC):
            e = c * _EC + k
            h1e = h1c[:, k * _W_B:(k + 1) * _W_B]
            h2e = jnp.maximum(
                jnp.dot(h1e, w2_ref[e], preferred_element_type=jnp.float32)
                .astype(jnp.bfloat16) + b2_ref[e:e + 1, :],
                jnp.bfloat16(0.0))
            cols.append(gate_bf[:, e:e + 1] * h2e)
    hcat = jnp.concatenate(cols, axis=1)
    y = jnp.dot(hcat, w3_ref[:], preferred_element_type=jnp.float32)
    y = y + jnp.dot(gate_bf, b3_ref[:],
                    preferred_element_type=jnp.float32)
    out_ref[:] = y


def kernel(x, W1, b1, W2, b2, W3, b3, Rw1, Rb1, Rw2, Rb2):
    w1_flat = W1.transpose(1, 0, 2).reshape(_D_IN, _E * _W_B)
    w1_flat = w1_flat.astype(jnp.bfloat16)
    b1_flat = b1.reshape(1, _E * _W_B).astype(jnp.bfloat16)
    w2_bf = W2.astype(jnp.bfloat16)
    b2_bf = b2.astype(jnp.bfloat16)
    w3_flat = W3.reshape(_E * _W_B, 1).astype(jnp.bfloat16)
    b3_bf = b3.astype(jnp.bfloat16)
    rw1_bf = Rw1.astype(jnp.bfloat16)
    rw2_bf = Rw2.astype(jnp.bfloat16)
    rb1_2d = Rb1.reshape(1, _W_R).astype(jnp.bfloat16)
    rb2_2d = Rb2.reshape(1, _E)

    grid = (_N_TOKENS // _BT,)
    full = lambda *shape: pl.BlockSpec(shape, lambda i: (0,) * len(shape))
    out = pl.pallas_call(
        _moe_kernel,
        grid=grid,
        in_specs=[
            pl.BlockSpec((_BT, _D_IN), lambda i: (i, 0)),
            full(_D_IN, _E * _W_B),
            full(1, _E * _W_B),
            full(_E, _W_B, _W_B),
            full(_E, _W_B),
            full(_E * _W_B, 1),
            full(_E, 1),
            full(_D_IN, _W_R),
            full(1, _W_R),
            full(_W_R, _E),
            full(1, _E),
        ],
        out_specs=pl.BlockSpec((_BT, 1), lambda i: (i, 0)),
        out_shape=jax.ShapeDtypeStruct((_N_TOKENS, 1), jnp.float32),
    )(x, w1_flat, b1_flat, w2_bf, b2_bf, w3_flat, b3_bf,
      rw1_bf, rb1_2d, rw2_bf, rb2_2d)
    return out
